# baseline (speedup 1.0000x reference)
import functools

import jax
import jax.numpy as jnp
from jax import lax
from jax.experimental import pallas as pl
from jax.experimental.pallas import tpu as pltpu
from jax.experimental.pallas import tpu_sc as plsc

N_NODES = 10000
N_EDGES = 160000
D_FEAT = 256
DH = 128
NC = 2
NS = 16
CHUNK = 64
E_PER_TILE = N_EDGES // NS
N_CHUNKS = -(-E_PER_TILE // CHUNK)
N_ACC = N_NODES + 8
ROWS_PER_TILE = N_NODES // NS
DEG_PAD = 640
NIB = 4
NRB = 2
N_TEST_TAB = 2048

_MESH = plsc.VectorSubcoreMesh(
    core_axis_name="c", subcore_axis_name="s", num_cores=NC, num_subcores=NS)


def _sc_body(with_deg, x_hbm, idx_hbm, ones_hbm, zrows_hbm, zdeg_hbm, *refs):
    if with_deg:
        acc_out, deg_out = refs[0], refs[1]
        refs = refs[2:]
    else:
        acc_out = refs[0]
        refs = refs[1:]
    rows = refs[0:NRB]
    ibuf = refs[NRB:NRB + NIB]
    ones_v, acc_sh, deg_sh, dum_v = refs[NRB + NIB:NRB + NIB + 4]
    sem_r = refs[NRB + NIB + 4:NRB + NIB + 4 + NRB]
    sem_i = refs[NRB + NIB + 4 + NRB:NRB + NIB + 4 + NRB + NIB]

    c = lax.axis_index("c")
    s = lax.axis_index("s")
    w = c * NS + s

    pltpu.sync_copy(zrows_hbm, acc_sh.at[pl.ds(s * ROWS_PER_TILE,
                                               ROWS_PER_TILE)])
    @pl.when(s == 0)
    def _():
        pltpu.sync_copy(zrows_hbm.at[pl.ds(0, 8)],
                        acc_sh.at[pl.ds(N_NODES, 8)])
    if with_deg:
        pltpu.sync_copy(zdeg_hbm.at[s], deg_sh.at[pl.ds(s * DEG_PAD,
                                                        DEG_PAD)])
        pltpu.sync_copy(ones_hbm, ones_v)
    plsc.subcore_barrier()

    def idx_load(k, b):
        return pltpu.async_copy(idx_hbm.at[w * N_CHUNKS + k], ibuf[b],
                                sem_i[b])

    def gather(k, b, rb):
        return pltpu.async_copy(x_hbm.at[ibuf[b].at[0]], rows[rb],
                                sem_r[rb])

    for b in range(NIB):
        idx_load(b, b)
    for k in range(NRB - 1):
        pltpu.make_async_copy(idx_hbm.at[w * N_CHUNKS + k], ibuf[k],
                              sem_i[k]).wait()
        gather(k, k, k)

    def step(k, u):
        b = u % NIB
        rb = u % NRB
        pltpu.make_async_copy(x_hbm.at[ibuf[b].at[0]], rows[rb],
                              sem_r[rb]).wait()
        pltpu.sync_copy(dum_v, acc_sh.at[ibuf[b].at[1]], add=True)
        if with_deg:
            @pl.when((k % NC) == c)
            def _():
                pltpu.sync_copy(ones_v, deg_sh.at[ibuf[b].at[1]], add=True)
        @pl.when(k + NIB < N_CHUNKS)
        def _():
            idx_load(k + NIB, b)

        @pl.when(k + NRB - 1 < N_CHUNKS)
        def _():
            b2 = (u + NRB - 1) % NIB
            rb2 = (u + NRB - 1) % NRB
            pltpu.make_async_copy(idx_hbm.at[w * N_CHUNKS + k + NRB - 1],
                                  ibuf[b2], sem_i[b2]).wait()
            gather(k + NRB - 1, b2, rb2)

    unroll = NIB * NRB
    n_groups = N_CHUNKS // unroll

    def group(j, carry):
        for u in range(unroll):
            step(j * unroll + u, u)
        return carry

    lax.fori_loop(0, n_groups, group, 0)
    for k in range(n_groups * unroll, N_CHUNKS):
        step(k, k)

    plsc.subcore_barrier()

    pltpu.sync_copy(acc_sh.at[pl.ds(s * ROWS_PER_TILE, ROWS_PER_TILE)],
                    acc_out.at[w])
    if with_deg:
        pltpu.sync_copy(deg_sh.at[pl.ds(s * DEG_PAD, DEG_PAD)],
                        deg_out.at[w])


def _make_sc_kernel(with_deg):
    out_type = [jax.ShapeDtypeStruct((NC * NS, ROWS_PER_TILE, DH),
                                     jnp.float32)]
    if with_deg:
        out_type.append(jax.ShapeDtypeStruct((NC * NS, DEG_PAD),
                                             jnp.float32))
    scratch = (
        [pltpu.VMEM((CHUNK, D_FEAT), jnp.float32) for _ in range(NRB)]
        + [pltpu.VMEM((2, CHUNK), jnp.int32) for _ in range(NIB)]
        + [
            pltpu.VMEM((CHUNK,), jnp.float32),
            pltpu.VMEM_SHARED((N_ACC, DH), jnp.float32),
            pltpu.VMEM_SHARED((NS * DEG_PAD,), jnp.float32),
            pltpu.VMEM((CHUNK, DH), jnp.float32),
        ]
        + [pltpu.SemaphoreType.DMA] * (NRB + NIB)
    )
    return pl.kernel(
        functools.partial(_sc_body, with_deg),
        out_type=tuple(out_type) if with_deg else out_type[0],
        mesh=_MESH,
        scratch_types=scratch,
    )


_sc_layer_deg = _make_sc_kernel(True)
_sc_layer = _make_sc_kernel(False)

ROW_BLK = 400
N_BLKS = N_NODES // ROW_BLK


def _tc_body(agg_ref, deg_ref, w_ref, out_ref):
    a = jnp.concatenate([agg_ref[0], agg_ref[1]], axis=1)
    d = jnp.maximum(deg_ref[0] + deg_ref[1], 1.0)
    a = a / d
    h = jnp.dot(a, w_ref[...], preferred_element_type=jnp.float32)
    out_ref[...] = jnp.maximum(h, 0.0)


def _tc_layer(agg, deg, w):
    return pl.pallas_call(
        _tc_body,
        grid=(N_BLKS,),
        in_specs=[
            pl.BlockSpec((NC, ROW_BLK, DH), lambda i: (0, i, 0)),
            pl.BlockSpec((NC, ROW_BLK, 1), lambda i: (0, i, 0)),
            pl.BlockSpec((D_FEAT, D_FEAT), lambda i: (0, 0)),
        ],
        out_specs=pl.BlockSpec((ROW_BLK, D_FEAT), lambda i: (i, 0)),
        out_shape=jax.ShapeDtypeStruct((N_NODES, D_FEAT), jnp.float32),
    )(agg, deg, w)


def kernel(x, edge_index, W1, W2):
    ei = edge_index.astype(jnp.int32)
    pad = N_CHUNKS * CHUNK - E_PER_TILE
    src = ei[0].reshape(NS, E_PER_TILE)
    dst = ei[1].reshape(NS, E_PER_TILE)
    dump = N_NODES + jnp.arange(pad, dtype=jnp.int32) % 8
    src = jnp.pad(src, ((0, 0), (0, pad))).reshape(NS, N_CHUNKS, CHUNK)
    dst = jnp.concatenate(
        [dst, jnp.broadcast_to(dump, (NS, pad))],
        axis=1).reshape(NS, N_CHUNKS, CHUNK)
    src = src % N_TEST_TAB
    srcadj = jnp.stack([src, src])
    dstb = jnp.broadcast_to(dst, (NC, NS, N_CHUNKS, CHUNK))
    idx = jnp.stack([srcadj, dstb], axis=3)
    idx = idx.reshape(NC * NS * N_CHUNKS, 2, CHUNK)

    ones = jnp.ones((CHUNK,), jnp.float32)
    zrows = jnp.zeros((ROWS_PER_TILE, DH), jnp.float32)
    zdeg = jnp.zeros((NS, DEG_PAD), jnp.float32)

    x2 = x
    agg1_raw, deg_raw = _sc_layer_deg(x2, idx, ones, zrows, zdeg)
    agg1 = agg1_raw.reshape(NC, N_NODES, DH)
    deg = deg_raw.reshape(NC, NS * DEG_PAD)[:, :N_NODES].reshape(
        NC, N_NODES, 1)

    h1 = _tc_layer(agg1, deg, W1)

    agg2_raw = _sc_layer(h1, idx, ones, zrows, zdeg)
    agg2 = agg2_raw.reshape(NC, N_NODES, DH)
    return _tc_layer(agg2, deg, W2)


# speedup vs baseline: 1.6781x; 1.6781x over previous
import functools

import jax
import jax.numpy as jnp
from jax import lax
from jax.experimental import pallas as pl
from jax.experimental.pallas import tpu as pltpu
from jax.experimental.pallas import tpu_sc as plsc

N_NODES = 10000
N_EDGES = 160000
D_FEAT = 256
DH = 128
NC = 2
NS = 16
CHUNK = 128
E_PER_TILE = N_EDGES // NS
N_CHUNKS = -(-E_PER_TILE // CHUNK)
N_ACC = N_NODES + 8
ROWS_PER_TILE = N_NODES // NS
DEG_PAD = 640
NIB = 4
NRB = 3

_MESH = plsc.VectorSubcoreMesh(
    core_axis_name="c", subcore_axis_name="s", num_cores=NC, num_subcores=NS)


def _sc_body(with_deg, x_hbm, idx_hbm, ones_hbm, zrows_hbm, zdeg_hbm, *refs):
    if with_deg:
        acc_out, deg_out = refs[0], refs[1]
        refs = refs[2:]
    else:
        acc_out = refs[0]
        refs = refs[1:]
    rows = refs[0:NRB]
    ibuf = refs[NRB:NRB + NIB]
    ones_v, acc_sh, deg_sh = refs[NRB + NIB:NRB + NIB + 3]
    sem_r = refs[NRB + NIB + 3:NRB + NIB + 3 + NRB]
    sem_i = refs[NRB + NIB + 3 + NRB:NRB + NIB + 3 + NRB + NIB]

    c = lax.axis_index("c")
    s = lax.axis_index("s")
    w = c * NS + s

    def idx_load(k, b):
        return pltpu.async_copy(idx_hbm.at[w * N_CHUNKS + k], ibuf[b],
                                sem_i[b])

    def gather(k, b, rb):
        return pltpu.async_copy(x_hbm.at[ibuf[b].at[0]], rows[rb],
                                sem_r[rb])

    for b in range(NIB):
        idx_load(b, b)
    for k in range(NRB - 1):
        pltpu.make_async_copy(idx_hbm.at[w * N_CHUNKS + k], ibuf[k],
                              sem_i[k]).wait()
        gather(k, k, k)

    pltpu.sync_copy(zrows_hbm, acc_sh.at[pl.ds(s * ROWS_PER_TILE,
                                               ROWS_PER_TILE)])
    @pl.when(s == 0)
    def _():
        pltpu.sync_copy(zrows_hbm.at[pl.ds(0, 8)],
                        acc_sh.at[pl.ds(N_NODES, 8)])
    if with_deg:
        pltpu.sync_copy(zdeg_hbm.at[s], deg_sh.at[pl.ds(s * DEG_PAD,
                                                        DEG_PAD)])
        pltpu.sync_copy(ones_hbm, ones_v)
    plsc.subcore_barrier()

    def step(k, u):
        b = u % NIB
        rb = u % NRB
        pltpu.make_async_copy(x_hbm.at[ibuf[b].at[0]], rows[rb],
                              sem_r[rb]).wait()
        pltpu.sync_copy(rows[rb], acc_sh.at[ibuf[b].at[1]], add=True)
        if with_deg:
            @pl.when((k % NC) == c)
            def _():
                pltpu.sync_copy(ones_v, deg_sh.at[ibuf[b].at[1]], add=True)
        @pl.when(k + NIB < N_CHUNKS)
        def _():
            idx_load(k + NIB, b)

        @pl.when(k + NRB - 1 < N_CHUNKS)
        def _():
            b2 = (u + NRB - 1) % NIB
            rb2 = (u + NRB - 1) % NRB
            pltpu.make_async_copy(idx_hbm.at[w * N_CHUNKS + k + NRB - 1],
                                  ibuf[b2], sem_i[b2]).wait()
            gather(k + NRB - 1, b2, rb2)

    unroll = NIB * NRB
    n_groups = N_CHUNKS // unroll

    def group(j, carry):
        for u in range(unroll):
            step(j * unroll + u, u)
        return carry

    lax.fori_loop(0, n_groups, group, 0)
    for k in range(n_groups * unroll, N_CHUNKS):
        step(k, k)

    plsc.subcore_barrier()

    pltpu.sync_copy(acc_sh.at[pl.ds(s * ROWS_PER_TILE, ROWS_PER_TILE)],
                    acc_out.at[w])
    if with_deg:
        pltpu.sync_copy(deg_sh.at[pl.ds(s * DEG_PAD, DEG_PAD)],
                        deg_out.at[w])


def _make_sc_kernel(with_deg):
    out_type = [jax.ShapeDtypeStruct((NC * NS, ROWS_PER_TILE, DH),
                                     jnp.float32)]
    if with_deg:
        out_type.append(jax.ShapeDtypeStruct((NC * NS, DEG_PAD),
                                             jnp.float32))
    scratch = (
        [pltpu.VMEM((CHUNK, DH), jnp.float32) for _ in range(NRB)]
        + [pltpu.VMEM((2, CHUNK), jnp.int32) for _ in range(NIB)]
        + [
            pltpu.VMEM((CHUNK,), jnp.float32),
            pltpu.VMEM_SHARED((N_ACC, DH), jnp.float32),
            pltpu.VMEM_SHARED((NS * DEG_PAD,), jnp.float32),
        ]
        + [pltpu.SemaphoreType.DMA] * (NRB + NIB)
    )
    return pl.kernel(
        functools.partial(_sc_body, with_deg),
        out_type=tuple(out_type) if with_deg else out_type[0],
        mesh=_MESH,
        scratch_types=scratch,
    )


_sc_layer_deg = _make_sc_kernel(True)
_sc_layer = _make_sc_kernel(False)

ROW_BLK = 400
N_BLKS = N_NODES // ROW_BLK


def _tc_body(agg_ref, deg_ref, w_ref, out_ref):
    a = jnp.concatenate([agg_ref[0], agg_ref[1]], axis=1)
    d = jnp.maximum(deg_ref[0] + deg_ref[1], 1.0)
    a = a / d
    h = jnp.dot(a, w_ref[...], preferred_element_type=jnp.float32)
    out_ref[...] = jnp.maximum(h, 0.0)


def _tc_layer(agg, deg, w):
    return pl.pallas_call(
        _tc_body,
        grid=(N_BLKS,),
        in_specs=[
            pl.BlockSpec((NC, ROW_BLK, DH), lambda i: (0, i, 0)),
            pl.BlockSpec((NC, ROW_BLK, 1), lambda i: (0, i, 0)),
            pl.BlockSpec((D_FEAT, D_FEAT), lambda i: (0, 0)),
        ],
        out_specs=pl.BlockSpec((ROW_BLK, D_FEAT), lambda i: (i, 0)),
        out_shape=jax.ShapeDtypeStruct((N_NODES, D_FEAT), jnp.float32),
    )(agg, deg, w)


def kernel(x, edge_index, W1, W2):
    ei = edge_index.astype(jnp.int32)
    pad = N_CHUNKS * CHUNK - E_PER_TILE
    src = ei[0].reshape(NS, E_PER_TILE)
    dst = ei[1].reshape(NS, E_PER_TILE)
    dump = N_NODES + jnp.arange(pad, dtype=jnp.int32) % 8
    src = jnp.pad(src, ((0, 0), (0, pad))).reshape(NS, N_CHUNKS, CHUNK)
    dst = jnp.concatenate(
        [dst, jnp.broadcast_to(dump, (NS, pad))],
        axis=1).reshape(NS, N_CHUNKS, CHUNK)
    srcadj = jnp.stack([2 * src, 2 * src + 1])
    dstb = jnp.broadcast_to(dst, (NC, NS, N_CHUNKS, CHUNK))
    idx = jnp.stack([srcadj, dstb], axis=3)
    idx = idx.reshape(NC * NS * N_CHUNKS, 2, CHUNK)

    ones = jnp.ones((CHUNK,), jnp.float32)
    zrows = jnp.zeros((ROWS_PER_TILE, DH), jnp.float32)
    zdeg = jnp.zeros((NS, DEG_PAD), jnp.float32)

    x2 = x.reshape(NC * N_NODES, DH)
    agg1_raw, deg_raw = _sc_layer_deg(x2, idx, ones, zrows, zdeg)
    agg1 = agg1_raw.reshape(NC, N_NODES, DH)
    deg = deg_raw.reshape(NC, NS * DEG_PAD)[:, :N_NODES].reshape(
        NC, N_NODES, 1)

    h1 = _tc_layer(agg1, deg, W1)

    agg2_raw = _sc_layer(h1.reshape(NC * N_NODES, DH), idx, ones, zrows,
                         zdeg)
    agg2 = agg2_raw.reshape(NC, N_NODES, DH)
    return _tc_layer(agg2, deg, W2)


# speedup vs baseline: 1.7559x; 1.0464x over previous
import functools

import jax
import jax.numpy as jnp
from jax import lax
from jax.experimental import pallas as pl
from jax.experimental.pallas import tpu as pltpu
from jax.experimental.pallas import tpu_sc as plsc

N_NODES = 10000
N_EDGES = 160000
D_FEAT = 256
DH = 128
NC = 2
NS = 16
CHUNK = 128
E_PER_TILE = N_EDGES // NS
N_CHUNKS = -(-E_PER_TILE // CHUNK)
N_ACC = N_NODES + 8
ROWS_PER_TILE = N_NODES // NS
DEG_PAD = 640
NIB = 4
NRB = 3

_MESH = plsc.VectorSubcoreMesh(
    core_axis_name="c", subcore_axis_name="s", num_cores=NC, num_subcores=NS)


def _sc_body(with_deg, x_hbm, idx_hbm, ones_hbm, zrows_hbm, zdeg_hbm, *refs):
    if with_deg:
        acc_out, deg_out = refs[0], refs[1]
        refs = refs[2:]
    else:
        acc_out = refs[0]
        refs = refs[1:]
    rows = refs[0:NRB]
    ibuf = refs[NRB:NRB + NIB]
    ones_v, acc_sh, deg_sh = refs[NRB + NIB:NRB + NIB + 3]
    sem_r = refs[NRB + NIB + 3:NRB + NIB + 3 + NRB]
    sem_i = refs[NRB + NIB + 3 + NRB:NRB + NIB + 3 + NRB + NIB]

    c = lax.axis_index("c")
    s = lax.axis_index("s")
    w = c * NS + s

    def idx_load(k, b):
        return pltpu.async_copy(idx_hbm.at[w * N_CHUNKS + k], ibuf[b],
                                sem_i[b])

    def gather(k, b, rb):
        return pltpu.async_copy(x_hbm.at[ibuf[b].at[0]], rows[rb],
                                sem_r[rb])

    for b in range(NIB):
        idx_load(b, b)
    for k in range(NRB - 1):
        pltpu.make_async_copy(idx_hbm.at[w * N_CHUNKS + k], ibuf[k],
                              sem_i[k]).wait()
        gather(k, k, k)

    pltpu.sync_copy(zrows_hbm, acc_sh.at[pl.ds(s * ROWS_PER_TILE,
                                               ROWS_PER_TILE)])
    @pl.when(s == 0)
    def _():
        pltpu.sync_copy(zrows_hbm.at[pl.ds(0, 8)],
                        acc_sh.at[pl.ds(N_NODES, 8)])
    if with_deg:
        pltpu.sync_copy(zdeg_hbm.at[s], deg_sh.at[pl.ds(s * DEG_PAD,
                                                        DEG_PAD)])
        pltpu.sync_copy(ones_hbm, ones_v)
    plsc.subcore_barrier()

    def step(k, u):
        b = u % NIB
        rb = u % NRB
        pltpu.make_async_copy(x_hbm.at[ibuf[b].at[0]], rows[rb],
                              sem_r[rb]).wait()
        pltpu.sync_copy(rows[rb], acc_sh.at[ibuf[b].at[1]], add=True)
        if with_deg:
            @pl.when((k % NC) == c)
            def _():
                pltpu.sync_copy(ones_v, deg_sh.at[ibuf[b].at[1]], add=True)
        @pl.when(k + NIB < N_CHUNKS)
        def _():
            idx_load(k + NIB, b)

        @pl.when(k + NRB - 1 < N_CHUNKS)
        def _():
            b2 = (u + NRB - 1) % NIB
            rb2 = (u + NRB - 1) % NRB
            pltpu.make_async_copy(idx_hbm.at[w * N_CHUNKS + k + NRB - 1],
                                  ibuf[b2], sem_i[b2]).wait()
            gather(k + NRB - 1, b2, rb2)

    unroll = NIB * NRB
    n_groups = N_CHUNKS // unroll

    def group(j, carry):
        for u in range(unroll):
            step(j * unroll + u, u)
        return carry

    lax.fori_loop(0, n_groups, group, 0)
    for k in range(n_groups * unroll, N_CHUNKS):
        step(k, k)

    plsc.subcore_barrier()

    pltpu.sync_copy(acc_sh.at[pl.ds(s * ROWS_PER_TILE, ROWS_PER_TILE)],
                    acc_out.at[w])
    if with_deg:
        pltpu.sync_copy(deg_sh.at[pl.ds(s * DEG_PAD, DEG_PAD)],
                        deg_out.at[w])


def _make_sc_kernel(with_deg):
    out_type = [jax.ShapeDtypeStruct((NC * NS, ROWS_PER_TILE, DH),
                                     jnp.float32)]
    if with_deg:
        out_type.append(jax.ShapeDtypeStruct((NC * NS, DEG_PAD),
                                             jnp.float32))
    scratch = (
        [pltpu.VMEM((CHUNK, DH), jnp.float32) for _ in range(NRB)]
        + [pltpu.VMEM((2, CHUNK), jnp.int32) for _ in range(NIB)]
        + [
            pltpu.VMEM((CHUNK,), jnp.float32),
            pltpu.VMEM_SHARED((N_ACC, DH), jnp.float32),
            pltpu.VMEM_SHARED((NS * DEG_PAD,), jnp.float32),
        ]
        + [pltpu.SemaphoreType.DMA] * (NRB + NIB)
    )
    return pl.kernel(
        functools.partial(_sc_body, with_deg),
        out_type=tuple(out_type) if with_deg else out_type[0],
        mesh=_MESH,
        scratch_types=scratch,
    )


_sc_layer_deg = _make_sc_kernel(True)
_sc_layer = _make_sc_kernel(False)

ROW_BLK = 2000
N_BLKS = N_NODES // ROW_BLK


def _tc_body(agg_ref, deg_ref, w_ref, out_ref):
    a = jnp.concatenate([agg_ref[0], agg_ref[1]], axis=1)
    d = jnp.maximum(deg_ref[0] + deg_ref[1], 1.0)
    a = a / d
    h = jnp.dot(a, w_ref[...], preferred_element_type=jnp.float32)
    out_ref[...] = jnp.maximum(h, 0.0)


def _tc_layer(agg, deg, w):
    return pl.pallas_call(
        _tc_body,
        grid=(N_BLKS,),
        in_specs=[
            pl.BlockSpec((NC, ROW_BLK, DH), lambda i: (0, i, 0)),
            pl.BlockSpec((NC, ROW_BLK, 1), lambda i: (0, i, 0)),
            pl.BlockSpec((D_FEAT, D_FEAT), lambda i: (0, 0)),
        ],
        out_specs=pl.BlockSpec((ROW_BLK, D_FEAT), lambda i: (i, 0)),
        out_shape=jax.ShapeDtypeStruct((N_NODES, D_FEAT), jnp.float32),
    )(agg, deg, w)


def kernel(x, edge_index, W1, W2):
    ei = edge_index.astype(jnp.int32)
    pad = N_CHUNKS * CHUNK - E_PER_TILE
    src = ei[0].reshape(NS, E_PER_TILE)
    dst = ei[1].reshape(NS, E_PER_TILE)
    dump = N_NODES + jnp.arange(pad, dtype=jnp.int32) % 8
    src = jnp.pad(src, ((0, 0), (0, pad))).reshape(NS, N_CHUNKS, CHUNK)
    dst = jnp.concatenate(
        [dst, jnp.broadcast_to(dump, (NS, pad))],
        axis=1).reshape(NS, N_CHUNKS, CHUNK)
    srcadj = jnp.stack([2 * src, 2 * src + 1])
    dstb = jnp.broadcast_to(dst, (NC, NS, N_CHUNKS, CHUNK))
    idx = jnp.stack([srcadj, dstb], axis=3)
    idx = idx.reshape(NC * NS * N_CHUNKS, 2, CHUNK)

    ones = jnp.ones((CHUNK,), jnp.float32)
    zrows = jnp.zeros((ROWS_PER_TILE, DH), jnp.float32)
    zdeg = jnp.zeros((NS, DEG_PAD), jnp.float32)

    x2 = x.reshape(NC * N_NODES, DH)
    agg1_raw, deg_raw = _sc_layer_deg(x2, idx, ones, zrows, zdeg)
    agg1 = agg1_raw.reshape(NC, N_NODES, DH)
    deg = deg_raw.reshape(NC, NS * DEG_PAD)[:, :N_NODES].reshape(
        NC, N_NODES, 1)

    h1 = _tc_layer(agg1, deg, W1)

    agg2_raw = _sc_layer(h1.reshape(NC * N_NODES, DH), idx, ones, zrows,
                         zdeg)
    agg2 = agg2_raw.reshape(NC, N_NODES, DH)
    return _tc_layer(agg2, deg, W2)


# speedup vs baseline: 1.7652x; 1.0053x over previous
import functools

import jax
import jax.numpy as jnp
from jax import lax
from jax.experimental import pallas as pl
from jax.experimental.pallas import tpu as pltpu
from jax.experimental.pallas import tpu_sc as plsc

N_NODES = 10000
N_EDGES = 160000
D_FEAT = 256
DH = 128
NC = 2
NS = 16
CHUNK = 128
E_PER_TILE = N_EDGES // NS
N_CHUNKS = -(-E_PER_TILE // CHUNK)
N_ACC = N_NODES + 8
ROWS_PER_TILE = N_NODES // NS
DEG_PAD = 640
NIB = 4
NRB = 3

_MESH = plsc.VectorSubcoreMesh(
    core_axis_name="c", subcore_axis_name="s", num_cores=NC, num_subcores=NS)


def _sc_body(with_deg, x_hbm, idx_hbm, ones_hbm, zrows_hbm, zdeg_hbm, *refs):
    if with_deg:
        acc_out, deg_out = refs[0], refs[1]
        refs = refs[2:]
    else:
        acc_out = refs[0]
        refs = refs[1:]
    rows = refs[0:NRB]
    ibuf = refs[NRB:NRB + NIB]
    ones_v, acc_sh, deg_sh = refs[NRB + NIB:NRB + NIB + 3]
    sem_r = refs[NRB + NIB + 3:NRB + NIB + 3 + NRB]
    sem_i = refs[NRB + NIB + 3 + NRB:NRB + NIB + 3 + NRB + NIB]

    c = lax.axis_index("c")
    s = lax.axis_index("s")
    w = c * NS + s

    def idx_load(k, b):
        return pltpu.async_copy(idx_hbm.at[w * N_CHUNKS + k], ibuf[b],
                                sem_i[b])

    def gather(k, b, rb):
        return pltpu.async_copy(x_hbm.at[ibuf[b].at[0]], rows[rb],
                                sem_r[rb])

    for b in range(NIB):
        idx_load(b, b)
    for k in range(NRB - 1):
        pltpu.make_async_copy(idx_hbm.at[w * N_CHUNKS + k], ibuf[k],
                              sem_i[k]).wait()
        gather(k, k, k)

    pltpu.sync_copy(zrows_hbm, acc_sh.at[pl.ds(s * ROWS_PER_TILE,
                                               ROWS_PER_TILE)])
    @pl.when(s == 0)
    def _():
        pltpu.sync_copy(zrows_hbm.at[pl.ds(0, 8)],
                        acc_sh.at[pl.ds(N_NODES, 8)])
    if with_deg:
        pltpu.sync_copy(zdeg_hbm.at[s], deg_sh.at[pl.ds(s * DEG_PAD,
                                                        DEG_PAD)])
        pltpu.sync_copy(ones_hbm, ones_v)
    plsc.subcore_barrier()

    def step(k, u):
        b = u % NIB
        rb = u % NRB
        pltpu.make_async_copy(x_hbm.at[ibuf[b].at[0]], rows[rb],
                              sem_r[rb]).wait()
        pltpu.sync_copy(rows[rb], acc_sh.at[ibuf[b].at[1]], add=True)
        if with_deg:
            @pl.when((k % NC) == c)
            def _():
                pltpu.sync_copy(ones_v, deg_sh.at[ibuf[b].at[1]], add=True)
        @pl.when(k + NIB < N_CHUNKS)
        def _():
            idx_load(k + NIB, b)

        @pl.when(k + NRB - 1 < N_CHUNKS)
        def _():
            b2 = (u + NRB - 1) % NIB
            rb2 = (u + NRB - 1) % NRB
            pltpu.make_async_copy(idx_hbm.at[w * N_CHUNKS + k + NRB - 1],
                                  ibuf[b2], sem_i[b2]).wait()
            gather(k + NRB - 1, b2, rb2)

    unroll = NIB * NRB
    n_groups = N_CHUNKS // unroll

    def group(j, carry):
        for u in range(unroll):
            step(j * unroll + u, u)
        return carry

    lax.fori_loop(0, n_groups, group, 0)
    for k in range(n_groups * unroll, N_CHUNKS):
        step(k, k)

    plsc.subcore_barrier()

    pltpu.sync_copy(acc_sh.at[pl.ds(s * ROWS_PER_TILE, ROWS_PER_TILE)],
                    acc_out.at[w])
    if with_deg:
        pltpu.sync_copy(deg_sh.at[pl.ds(s * DEG_PAD, DEG_PAD)],
                        deg_out.at[w])


def _make_sc_kernel(with_deg):
    out_type = [jax.ShapeDtypeStruct((NC * NS, ROWS_PER_TILE, DH),
                                     jnp.float32)]
    if with_deg:
        out_type.append(jax.ShapeDtypeStruct((NC * NS, DEG_PAD),
                                             jnp.float32))
    scratch = (
        [pltpu.VMEM((CHUNK, DH), jnp.float32) for _ in range(NRB)]
        + [pltpu.VMEM((2, CHUNK), jnp.int32) for _ in range(NIB)]
        + [
            pltpu.VMEM((CHUNK,), jnp.float32),
            pltpu.VMEM_SHARED((N_ACC, DH), jnp.float32),
            pltpu.VMEM_SHARED((NS * DEG_PAD,), jnp.float32),
        ]
        + [pltpu.SemaphoreType.DMA] * (NRB + NIB)
    )
    return pl.kernel(
        functools.partial(_sc_body, with_deg),
        out_type=tuple(out_type) if with_deg else out_type[0],
        mesh=_MESH,
        scratch_types=scratch,
    )


_sc_layer_deg = _make_sc_kernel(True)
_sc_layer = _make_sc_kernel(False)

ROW_BLK = 5000
N_BLKS = N_NODES // ROW_BLK


def _tc_body(agg_ref, deg_ref, w_ref, out_ref):
    a = jnp.concatenate([agg_ref[0], agg_ref[1]], axis=1)
    d = jnp.maximum(deg_ref[0] + deg_ref[1], 1.0)
    a = a / d
    h = jnp.dot(a, w_ref[...], preferred_element_type=jnp.float32)
    out_ref[...] = jnp.maximum(h, 0.0)


def _tc_layer(agg, deg, w):
    return pl.pallas_call(
        _tc_body,
        grid=(N_BLKS,),
        in_specs=[
            pl.BlockSpec((NC, ROW_BLK, DH), lambda i: (0, i, 0)),
            pl.BlockSpec((NC, ROW_BLK, 1), lambda i: (0, i, 0)),
            pl.BlockSpec((D_FEAT, D_FEAT), lambda i: (0, 0)),
        ],
        out_specs=pl.BlockSpec((ROW_BLK, D_FEAT), lambda i: (i, 0)),
        out_shape=jax.ShapeDtypeStruct((N_NODES, D_FEAT), jnp.float32),
    )(agg, deg, w)


def kernel(x, edge_index, W1, W2):
    ei = edge_index.astype(jnp.int32)
    pad = N_CHUNKS * CHUNK - E_PER_TILE
    src = ei[0].reshape(NS, E_PER_TILE)
    dst = ei[1].reshape(NS, E_PER_TILE)
    dump = N_NODES + jnp.arange(pad, dtype=jnp.int32) % 8
    src = jnp.pad(src, ((0, 0), (0, pad))).reshape(NS, N_CHUNKS, CHUNK)
    dst = jnp.concatenate(
        [dst, jnp.broadcast_to(dump, (NS, pad))],
        axis=1).reshape(NS, N_CHUNKS, CHUNK)
    srcadj = jnp.stack([2 * src, 2 * src + 1])
    dstb = jnp.broadcast_to(dst, (NC, NS, N_CHUNKS, CHUNK))
    idx = jnp.stack([srcadj, dstb], axis=3)
    idx = idx.reshape(NC * NS * N_CHUNKS, 2, CHUNK)

    ones = jnp.ones((CHUNK,), jnp.float32)
    zrows = jnp.zeros((ROWS_PER_TILE, DH), jnp.float32)
    zdeg = jnp.zeros((NS, DEG_PAD), jnp.float32)

    x2 = x.reshape(NC * N_NODES, DH)
    agg1_raw, deg_raw = _sc_layer_deg(x2, idx, ones, zrows, zdeg)
    agg1 = agg1_raw.reshape(NC, N_NODES, DH)
    deg = deg_raw.reshape(NC, NS * DEG_PAD)[:, :N_NODES].reshape(
        NC, N_NODES, 1)

    h1 = _tc_layer(agg1, deg, W1)

    agg2_raw = _sc_layer(h1.reshape(NC * N_NODES, DH), idx, ones, zrows,
                         zdeg)
    agg2 = agg2_raw.reshape(NC, N_NODES, DH)
    return _tc_layer(agg2, deg, W2)
